# Initial kernel scaffold; baseline (speedup 1.0000x reference)
import jax
import jax.numpy as jnp
from jax.experimental import pallas as pl


def kernel(x, table, pos_encoding):
    raise NotImplementedError("write your pallas kernel here")


# speedup vs baseline: 1.7349x; 1.7349x over previous
import functools

import jax
import jax.numpy as jnp
from jax import lax
from jax.experimental import pallas as pl
from jax.experimental.pallas import tpu as pltpu
from jax.experimental.pallas import tpu_sc as plsc

VOCAB = 100000
D = 128
SENT = 50
BATCH = 4096
B = BATCH * SENT
NW = 32
ROWS_PER_W = B // NW
SENT_PER_CHUNK = 8
CHUNK = SENT_PER_CHUNK * SENT
NCHUNK = ROWS_PER_W // CHUNK
SCALE = float(D) ** 0.5
NLANE = D // 16


def _sc_body(idx_hbm, table_hbm, pe_hbm, out_hbm,
             idx_v, pe_v, buf0, buf1, sem0, sem1, osem):
    wid = lax.axis_index("s") * 2 + lax.axis_index("c")
    base = wid * ROWS_PER_W

    pltpu.sync_copy(idx_hbm.at[pl.ds(base, ROWS_PER_W)], idx_v)
    pltpu.sync_copy(pe_hbm, pe_v)

    bufs = (buf0, buf1)
    sems = (sem0, sem1)

    def start_gather(k, buf, sem):
        return pltpu.async_copy(
            table_hbm.at[idx_v.at[pl.ds(k * CHUNK, CHUNK)]], buf, sem)

    def compute(buf):
        def body_s(s, _):
            def body_t(t, _):
                r = t * SENT + s
                for v in range(NLANE):
                    sl = pl.ds(v * 16, 16)
                    buf[r, sl] = buf[r, sl] * SCALE + pe_v[s, sl]
                return 0
            lax.fori_loop(0, SENT_PER_CHUNK, body_t, 0, unroll=False)
            return 0
        lax.fori_loop(0, SENT, body_s, 0, unroll=False)

    pending = start_gather(0, bufs[0], sems[0])
    for k in range(NCHUNK):
        b = k % 2
        pending.wait()
        if k + 1 < NCHUNK:
            pending = start_gather(k + 1, bufs[1 - b], sems[1 - b])
        compute(bufs[b])
        pltpu.sync_copy(bufs[b], out_hbm.at[pl.ds(base + k * CHUNK, CHUNK)])


@jax.jit
def _run(idx_flat, table, pe):
    k = pl.kernel(
        _sc_body,
        out_type=jax.ShapeDtypeStruct((B, D), jnp.float32),
        mesh=plsc.VectorSubcoreMesh(core_axis_name="c", subcore_axis_name="s"),
        scratch_types=[
            pltpu.VMEM((ROWS_PER_W,), jnp.int32),
            pltpu.VMEM((SENT, D), jnp.float32),
            pltpu.VMEM((CHUNK, D), jnp.float32),
            pltpu.VMEM((CHUNK, D), jnp.float32),
            pltpu.SemaphoreType.DMA,
            pltpu.SemaphoreType.DMA,
            pltpu.SemaphoreType.DMA,
        ],
    )
    return k(idx_flat, table, pe)


def kernel(x, table, pos_encoding):
    idx_flat = x.reshape(-1).astype(jnp.int32)
    out = _run(idx_flat, table, pos_encoding)
    return out.reshape(BATCH, SENT, D)


# speedup vs baseline: 3.2549x; 1.8762x over previous
import functools

import jax
import jax.numpy as jnp
from jax import lax
from jax.experimental import pallas as pl
from jax.experimental.pallas import tpu as pltpu
from jax.experimental.pallas import tpu_sc as plsc

VOCAB = 100000
D = 128
SENT = 50
BATCH = 4096
B = BATCH * SENT
NW = 32
ROWS_PER_W = B // NW
SENT_PER_CHUNK = 8
CHUNK = SENT_PER_CHUNK * SENT
NCHUNK = ROWS_PER_W // CHUNK
SCALE = float(D) ** 0.5
NLANE = D // 16


def _sc_body(idx_hbm, table_hbm, pe_hbm, out_hbm,
             idx_v, pe_v, buf0, buf1, sem0, sem1, osem0, osem1):
    wid = lax.axis_index("s") * 2 + lax.axis_index("c")
    base = wid * ROWS_PER_W

    pltpu.sync_copy(idx_hbm.at[pl.ds(base, ROWS_PER_W)], idx_v)
    pltpu.sync_copy(pe_hbm, pe_v)

    bufs = (buf0, buf1)
    sems = (sem0, sem1)

    def start_gather(k, buf, sem):
        return pltpu.async_copy(
            table_hbm.at[idx_v.at[pl.ds(k * CHUNK, CHUNK)]], buf, sem)

    def compute(buf):
        def body_s(s, _):
            pe_vecs = [pe_v[s, pl.ds(v * 16, 16)] for v in range(NLANE)]
            for t in range(SENT_PER_CHUNK):
                r = t * SENT + s
                for v in range(NLANE):
                    sl = pl.ds(v * 16, 16)
                    buf[r, sl] = buf[r, sl] * SCALE + pe_vecs[v]
            return 0
        lax.fori_loop(0, SENT, body_s, 0, unroll=False)

    def start_put(k, buf, sem):
        return pltpu.async_copy(
            buf, out_hbm.at[pl.ds(base + k * CHUNK, CHUNK)], sem)

    osems = (osem0, osem1)
    pending = start_gather(0, bufs[0], sems[0])
    out_pending = [None, None]
    for k in range(NCHUNK):
        b = k % 2
        pending.wait()
        if k + 1 < NCHUNK:
            if out_pending[1 - b] is not None:
                out_pending[1 - b].wait()
            pending = start_gather(k + 1, bufs[1 - b], sems[1 - b])
        compute(bufs[b])
        out_pending[b] = start_put(k, bufs[b], osems[b])
    out_pending[(NCHUNK - 1) % 2].wait()


@jax.jit
def _run(idx_flat, table, pe):
    k = pl.kernel(
        _sc_body,
        out_type=jax.ShapeDtypeStruct((B, D), jnp.float32),
        mesh=plsc.VectorSubcoreMesh(core_axis_name="c", subcore_axis_name="s"),
        scratch_types=[
            pltpu.VMEM((ROWS_PER_W,), jnp.int32),
            pltpu.VMEM((SENT, D), jnp.float32),
            pltpu.VMEM((CHUNK, D), jnp.float32),
            pltpu.VMEM((CHUNK, D), jnp.float32),
            pltpu.SemaphoreType.DMA,
            pltpu.SemaphoreType.DMA,
            pltpu.SemaphoreType.DMA,
            pltpu.SemaphoreType.DMA,
        ],
    )
    return k(idx_flat, table, pe)


def kernel(x, table, pos_encoding):
    idx_flat = x.reshape(-1).astype(jnp.int32)
    out = _run(idx_flat, table, pos_encoding)
    return out.reshape(BATCH, SENT, D)
